# baseline (speedup 1.0000x reference)
import functools

import jax
import jax.numpy as jnp
from jax import lax
from jax.experimental import pallas as pl
from jax.experimental.pallas import tpu as pltpu
from jax.experimental.pallas import tpu_sc as plsc

N = 10000
E = 160000
DIN = 256
H = 512
G = 128
NPAD = 10240
RB = 1024
NBLK = NPAD // RB
LOG2 = 0.6931471805599453

NSC = 2
NTILE = 16
EPAD = 163840
ECH = EPAD // (NTILE * 128)
RPT = NPAD // NTILE
GCOL = 128


def _relu(v):
    return jnp.maximum(v, 0.0)


def _bdot(a, b):
    return jnp.dot(a, b, preferred_element_type=jnp.float32)


def _bdotg(a, b, dn):
    return lax.dot_general(a, b, dn, preferred_element_type=jnp.float32)


def _make_mlp(nin):
    def body(*refs):
        grefs = refs[0:nin]
        wa_ref, ba_ref, wb_ref, bb_ref = refs[nin:nin + 4]
        outs = refs[nin + 4:]
        hpa = jnp.concatenate([g[...] for g in grefs], axis=1)
        t = _relu(_bdot(hpa, wa_ref[...]) + ba_ref[...])
        u = _relu(_bdot(t, wb_ref[...]) + bb_ref[...])
        for i, o in enumerate(outs):
            o[...] = u[:, i * GCOL:(i + 1) * GCOL]

    gspec = pl.BlockSpec((RB, GCOL), lambda i: (i, 0))
    K = nin * GCOL
    return pl.pallas_call(
        body,
        grid=(NBLK,),
        in_specs=[gspec] * nin + [
            pl.BlockSpec((K, H), lambda i: (0, 0)),
            pl.BlockSpec((1, H), lambda i: (0, 0)),
            pl.BlockSpec((H, H), lambda i: (0, 0)),
            pl.BlockSpec((1, H), lambda i: (0, 0)),
        ],
        out_specs=[gspec] * (H // GCOL),
        out_shape=[jax.ShapeDtypeStruct((NPAD, GCOL), jnp.float32)]
        * (H // GCOL),
    )


_mlp = {2: _make_mlp(2), 4: _make_mlp(4)}


def _gin_mlp(hpa_groups, Wa, ba, Wb, bb):
    return _mlp[len(hpa_groups)](*hpa_groups, Wa, ba.reshape(1, H), Wb,
                                 bb.reshape(1, H))


def _seg_body(*refs):
    b_ref = refs[0]
    hrefs = refs[1:13]
    o1_ref, o2_ref, o3_ref = refs[13:]

    @pl.when(pl.program_id(0) == 0)
    def _():
        o1_ref[...] = jnp.zeros_like(o1_ref)
        o2_ref[...] = jnp.zeros_like(o2_ref)
        o3_ref[...] = jnp.zeros_like(o3_ref)

    seg = b_ref[...]
    m = (seg == lax.broadcasted_iota(jnp.int32, (RB, G), 1)).astype(jnp.float32)
    dn = (((0,), (0,)), ((), ()))
    h1 = jnp.concatenate([g[...] for g in hrefs[0:4]], axis=1)
    h2 = jnp.concatenate([g[...] for g in hrefs[4:8]], axis=1)
    h3 = jnp.concatenate([g[...] for g in hrefs[8:12]], axis=1)
    o1_ref[...] += _bdotg(m, h1, dn)
    o2_ref[...] += _bdotg(m, h2, dn)
    o3_ref[...] += _bdotg(m, h3, dn)


def _segment_readout(batch_p, h1g, h2g, h3g):
    gspec = pl.BlockSpec((RB, GCOL), lambda i: (i, 0))
    return pl.pallas_call(
        _seg_body,
        grid=(NBLK,),
        in_specs=[pl.BlockSpec((RB, 1), lambda i: (i, 0))] + [gspec] * 12,
        out_specs=[pl.BlockSpec((G, H), lambda i: (0, 0))] * 3,
        out_shape=[jax.ShapeDtypeStruct((G, H), jnp.float32)] * 3,
    )(batch_p, *h1g, *h2g, *h3g)


def _gff_body(g1_ref, g2_ref, g3_ref, w0_ref, b0_ref, w1_ref, b1_ref,
              ws_ref, bs_ref, out_ref):
    z0 = (_bdot(g1_ref[...], w0_ref[0:H])
          + _bdot(g2_ref[...], w0_ref[H:2 * H])
          + _bdot(g3_ref[...], w0_ref[2 * H:3 * H])
          + b0_ref[...])
    blk = _relu(z0)
    blk2 = _relu(_bdot(blk, w1_ref[...])
                 + b1_ref[...])
    zs = (_bdot(g1_ref[...], ws_ref[0:H])
          + _bdot(g2_ref[...], ws_ref[H:2 * H])
          + _bdot(g3_ref[...], ws_ref[2 * H:3 * H])
          + bs_ref[...])
    out_ref[...] = blk2 + zs


def _global_ff(g1, g2, g3, gW0, gb0, gW1, gb1, gWs, gbs):
    return pl.pallas_call(
        _gff_body,
        out_shape=jax.ShapeDtypeStruct((G, H), jnp.float32),
    )(g1, g2, g3, gW0, gb0.reshape(1, H), gW1, gb1.reshape(1, H),
      gWs, gbs.reshape(1, H))


def _loss_body(*refs):
    b_ref = refs[0]
    hrefs = refs[1:13]
    (g_ref, w0_ref, b0_ref, w1_ref, b1_ref, ws_ref, bs_ref,
     pos_ref, neg_ref) = refs[13:]
    i = pl.program_id(0)

    @pl.when(i == 0)
    def _():
        pos_ref[...] = jnp.zeros_like(pos_ref)
        neg_ref[...] = jnp.zeros_like(neg_ref)

    h1 = jnp.concatenate([g[...] for g in hrefs[0:4]], axis=1)
    h2 = jnp.concatenate([g[...] for g in hrefs[4:8]], axis=1)
    h3 = jnp.concatenate([g[...] for g in hrefs[8:12]], axis=1)
    z0 = (_bdot(h1, w0_ref[0:H])
          + _bdot(h2, w0_ref[H:2 * H])
          + _bdot(h3, w0_ref[2 * H:3 * H])
          + b0_ref[...])
    blk = _relu(z0)
    blk2 = _relu(_bdot(blk, w1_ref[...])
                 + b1_ref[...])
    zs = (_bdot(h1, ws_ref[0:H])
          + _bdot(h2, ws_ref[H:2 * H])
          + _bdot(h3, ws_ref[2 * H:3 * H])
          + bs_ref[...])
    l_enc = blk2 + zs

    res = _bdotg(l_enc, g_ref[...], (((1,), (1,)), ((), ())))
    seg = b_ref[...]
    gid = lax.broadcasted_iota(jnp.int32, (RB, G), 1)
    posm = seg == gid
    rowid = lax.broadcasted_iota(jnp.int32, (RB, G), 0) + i * RB
    valid = rowid < N
    sp = jnp.maximum(-res, 0.0) + jnp.log1p(jnp.exp(-jnp.abs(res)))
    posv = jnp.where(posm, LOG2 - sp, 0.0)
    negv = jnp.where(jnp.logical_and(jnp.logical_not(posm), valid),
                     sp + res - LOG2, 0.0)
    pos_ref[...] += jnp.sum(posv).reshape(1, 1)
    neg_ref[...] += jnp.sum(negv).reshape(1, 1)


def _loss_stage(batch_p, h1g, h2g, h3g, g_enc, lW0, lb0, lW1, lb1, lWs, lbs):
    gspec = pl.BlockSpec((RB, GCOL), lambda i: (i, 0))
    pos, neg = pl.pallas_call(
        _loss_body,
        grid=(NBLK,),
        in_specs=[pl.BlockSpec((RB, 1), lambda i: (i, 0))] + [gspec] * 12 + [
            pl.BlockSpec((G, H), lambda i: (0, 0)),
            pl.BlockSpec((3 * H, H), lambda i: (0, 0)),
            pl.BlockSpec((1, H), lambda i: (0, 0)),
            pl.BlockSpec((H, H), lambda i: (0, 0)),
            pl.BlockSpec((1, H), lambda i: (0, 0)),
            pl.BlockSpec((3 * H, H), lambda i: (0, 0)),
            pl.BlockSpec((1, H), lambda i: (0, 0)),
        ],
        out_specs=[pl.BlockSpec((1, 1), lambda i: (0, 0))] * 2,
        out_shape=[jax.ShapeDtypeStruct((1, 1), jnp.float32)] * 2,
    )(batch_p, *h1g, *h2g, *h3g, g_enc, lW0, lb0.reshape(1, H), lW1,
      lb1.reshape(1, H), lWs, lbs.reshape(1, H))
    return pos, neg


IDXH = ECH // 2


def _make_sc_agg(ncg):
    gpc = ncg // NSC
    mesh = plsc.VectorSubcoreMesh(core_axis_name="c", subcore_axis_name="s")

    def body(*refs):
        hgs = refs[0:ncg]
        src_h = refs[ncg]
        dst_h = refs[ncg + 1]
        outs = refs[ncg + 2:2 * ncg + 2]
        (src_v, dst_v, buf_a, buf_b, agg_sh,
         gsem_a, gsem_b, ssem_a, ssem_b) = refs[2 * ncg + 2:]
        bufs = (buf_a, buf_b)
        gsems = (gsem_a, gsem_b)
        ssems = (ssem_a, ssem_b)
        c = lax.axis_index("c")
        s = lax.axis_index("s")
        base = s * RPT

        def gstart(gi, j, b):
            pltpu.async_copy(hgs[gi].at[src_v.at[j]], bufs[b], gsems[b])

        def gwait(gi, j, b):
            pltpu.make_async_copy(hgs[gi].at[src_v.at[j]], bufs[b],
                                  gsems[b]).wait()

        def sstart(j, b):
            pltpu.async_copy(bufs[b], agg_sh.at[dst_v.at[j]], ssems[b],
                             add=True)

        def swait(j, b):
            pltpu.make_async_copy(bufs[b], agg_sh.at[dst_v.at[j]],
                                  ssems[b]).wait()

        for gl in range(gpc):
            for ci in range(NSC):
                @pl.when(c == ci)
                def _(gi=ci * gpc + gl):
                    pltpu.sync_copy(hgs[gi].at[pl.ds(base, RPT)],
                                    agg_sh.at[pl.ds(base, RPT)])
            plsc.subcore_barrier()
            for ci in range(NSC):
                @pl.when(c == ci)
                def _(gi=ci * gpc + gl):
                    for half in range(2):
                        pltpu.sync_copy(src_h.at[s, pl.ds(half * IDXH, IDXH)],
                                        src_v)
                        pltpu.sync_copy(dst_h.at[s, pl.ds(half * IDXH, IDXH)],
                                        dst_v)
                        gstart(gi, 0, 0)

                        def step(jj, carry, gi=gi):
                            for b in range(2):
                                j = jj * 2 + b
                                gwait(gi, j, b)
                                sstart(j, b)

                                @pl.when(j + 1 < IDXH)
                                def _(b=b, j=j, gi=gi):
                                    @pl.when(j >= 1)
                                    def _():
                                        swait(j - 1, 1 - b)
                                    gstart(gi, j + 1, 1 - b)
                            return carry
                        lax.fori_loop(0, IDXH // 2, step, 0, unroll=False)
                        swait(IDXH - 2, IDXH % 2)
                        swait(IDXH - 1, (IDXH - 1) % 2)
            plsc.subcore_barrier()
            for ci in range(NSC):
                @pl.when(c == ci)
                def _(gi=ci * gpc + gl):
                    pltpu.sync_copy(agg_sh.at[pl.ds(base, RPT)],
                                    outs[gi].at[pl.ds(base, RPT)])

    return pl.kernel(
        body,
        mesh=mesh,
        out_type=[jax.ShapeDtypeStruct((NPAD, GCOL), jnp.float32)] * ncg,
        scratch_types=[
            pltpu.VMEM((IDXH, 128), jnp.int32),
            pltpu.VMEM((IDXH, 128), jnp.int32),
            pltpu.VMEM((128, GCOL), jnp.float32),
            pltpu.VMEM((128, GCOL), jnp.float32),
            pltpu.VMEM_SHARED((NPAD, GCOL), jnp.float32),
            pltpu.SemaphoreType.DMA,
            pltpu.SemaphoreType.DMA,
            pltpu.SemaphoreType.DMA,
            pltpu.SemaphoreType.DMA,
        ],
    )


_sc_agg = {2: _make_sc_agg(2), 4: _make_sc_agg(4)}


def _edge_agg(groups, srcp, dstp):
    return list(_sc_agg[len(groups)](*groups, srcp, dstp))


def kernel(x, edge_index, batch, W0a, b0a, W0b, b0b, W1a, b1a, W1b, b1b,
           W2a, b2a, W2b, b2b, lW0, lb0, lW1, lb1, lWs, lbs,
           gW0, gb0, gW1, gb1, gWs, gbs):
    src = edge_index[0]
    dst = edge_index[1]
    pad_e = EPAD - E
    srcp = jnp.concatenate(
        [src.astype(jnp.int32),
         N + (jnp.arange(pad_e, dtype=jnp.int32) % 8)]).reshape(NTILE, ECH, 128)
    dstp = jnp.concatenate(
        [dst.astype(jnp.int32),
         N + (jnp.arange(pad_e, dtype=jnp.int32) % (NPAD - N))]
    ).reshape(NTILE, ECH, 128)
    batch_p = jnp.pad(batch, (0, NPAD - N), constant_values=G)
    batch_p = batch_p.astype(jnp.int32).reshape(NPAD, 1)
    xg = [jnp.pad(x[:, i * GCOL:(i + 1) * GCOL], ((0, NPAD - N), (0, 0)))
          for i in range(DIN // GCOL)]

    h1 = _gin_mlp(_edge_agg(xg, srcp, dstp), W0a, b0a, W0b, b0b)
    h2 = _gin_mlp(_edge_agg(h1, srcp, dstp), W1a, b1a, W1b, b1b)
    h3 = _gin_mlp(_edge_agg(h2, srcp, dstp), W2a, b2a, W2b, b2b)

    g1, g2, g3 = _segment_readout(batch_p, h1, h2, h3)
    g_enc = _global_ff(g1, g2, g3, gW0, gb0, gW1, gb1, gWs, gbs)
    pos, neg = _loss_stage(batch_p, h1, h2, h3, g_enc,
                           lW0, lb0, lW1, lb1, lWs, lbs)
    e_pos = pos[0, 0] / jnp.float32(N)
    e_neg = neg[0, 0] / jnp.float32(N * (G - 1))
    return e_neg - e_pos


# speedup vs baseline: 1.0023x; 1.0023x over previous
import jax
import jax.numpy as jnp
from jax import lax
from jax.experimental import pallas as pl
from jax.experimental.pallas import tpu as pltpu
from jax.experimental.pallas import tpu_sc as plsc

N = 10000
E = 160000
DIN = 256
H = 512
G = 128
NPAD = 10240
RB = 1024
NBLK = NPAD // RB
LOG2 = 0.6931471805599453

NSC = 2
NTILE = 16
EPAD = 163840
ECH = EPAD // (NTILE * 128)
RPT = NPAD // NTILE
GCOL = 128


def _relu(v):
    return jnp.maximum(v, 0.0)


def _dot(a, b):
    return jnp.dot(a, b, preferred_element_type=jnp.float32)


def _dotg(a, b, dn):
    return lax.dot_general(a, b, dn, preferred_element_type=jnp.float32)


def _make_mlp(nin):
    def body(*refs):
        grefs = refs[0:nin]
        wa_ref, ba_ref, wb_ref, bb_ref = refs[nin:nin + 4]
        outs = refs[nin + 4:]
        hpa = jnp.concatenate([g[...] for g in grefs], axis=1)
        t = _relu(_dot(hpa, wa_ref[...]) + ba_ref[...])
        u = _relu(_dot(t, wb_ref[...]) + bb_ref[...])
        for i, o in enumerate(outs):
            o[...] = u[:, i * GCOL:(i + 1) * GCOL]

    gspec = pl.BlockSpec((RB, GCOL), lambda i: (i, 0))
    K = nin * GCOL
    return pl.pallas_call(
        body,
        grid=(NBLK,),
        in_specs=[gspec] * nin + [
            pl.BlockSpec((K, H), lambda i: (0, 0)),
            pl.BlockSpec((1, H), lambda i: (0, 0)),
            pl.BlockSpec((H, H), lambda i: (0, 0)),
            pl.BlockSpec((1, H), lambda i: (0, 0)),
        ],
        out_specs=[gspec] * (H // GCOL),
        out_shape=[jax.ShapeDtypeStruct((NPAD, GCOL), jnp.float32)]
        * (H // GCOL),
    )


_mlp = {2: _make_mlp(2), 4: _make_mlp(4)}


def _gin_mlp(hpa_groups, Wa, ba, Wb, bb):
    return _mlp[len(hpa_groups)](*hpa_groups, Wa, ba.reshape(1, H), Wb,
                                 bb.reshape(1, H))


def _seg_body(*refs):
    b_ref = refs[0]
    hrefs = refs[1:13]
    o1_ref, o2_ref, o3_ref = refs[13:]

    @pl.when(pl.program_id(0) == 0)
    def _():
        o1_ref[...] = jnp.zeros_like(o1_ref)
        o2_ref[...] = jnp.zeros_like(o2_ref)
        o3_ref[...] = jnp.zeros_like(o3_ref)

    seg = b_ref[...]
    m = (seg == lax.broadcasted_iota(jnp.int32, (RB, G), 1)).astype(jnp.float32)
    dn = (((0,), (0,)), ((), ()))
    h1 = jnp.concatenate([g[...] for g in hrefs[0:4]], axis=1)
    h2 = jnp.concatenate([g[...] for g in hrefs[4:8]], axis=1)
    h3 = jnp.concatenate([g[...] for g in hrefs[8:12]], axis=1)
    o1_ref[...] += _dotg(m, h1, dn)
    o2_ref[...] += _dotg(m, h2, dn)
    o3_ref[...] += _dotg(m, h3, dn)


def _segment_readout(batch_p, h1g, h2g, h3g):
    gspec = pl.BlockSpec((RB, GCOL), lambda i: (i, 0))
    return pl.pallas_call(
        _seg_body,
        grid=(NBLK,),
        in_specs=[pl.BlockSpec((RB, 1), lambda i: (i, 0))] + [gspec] * 12,
        out_specs=[pl.BlockSpec((G, H), lambda i: (0, 0))] * 3,
        out_shape=[jax.ShapeDtypeStruct((G, H), jnp.float32)] * 3,
    )(batch_p, *h1g, *h2g, *h3g)


def _gff_body(g1_ref, g2_ref, g3_ref, w0_ref, b0_ref, w1_ref, b1_ref,
              ws_ref, bs_ref, out_ref):
    z0 = (_dot(g1_ref[...], w0_ref[0:H])
          + _dot(g2_ref[...], w0_ref[H:2 * H])
          + _dot(g3_ref[...], w0_ref[2 * H:3 * H])
          + b0_ref[...])
    blk = _relu(z0)
    blk2 = _relu(_dot(blk, w1_ref[...])
                 + b1_ref[...])
    zs = (_dot(g1_ref[...], ws_ref[0:H])
          + _dot(g2_ref[...], ws_ref[H:2 * H])
          + _dot(g3_ref[...], ws_ref[2 * H:3 * H])
          + bs_ref[...])
    out_ref[...] = blk2 + zs


def _global_ff(g1, g2, g3, gW0, gb0, gW1, gb1, gWs, gbs):
    return pl.pallas_call(
        _gff_body,
        out_shape=jax.ShapeDtypeStruct((G, H), jnp.float32),
    )(g1, g2, g3, gW0, gb0.reshape(1, H), gW1, gb1.reshape(1, H),
      gWs, gbs.reshape(1, H))


def _loss_body(*refs):
    b_ref = refs[0]
    hrefs = refs[1:13]
    (g_ref, w0_ref, b0_ref, w1_ref, b1_ref, ws_ref, bs_ref,
     pos_ref, neg_ref) = refs[13:]
    i = pl.program_id(0)

    @pl.when(i == 0)
    def _():
        pos_ref[...] = jnp.zeros_like(pos_ref)
        neg_ref[...] = jnp.zeros_like(neg_ref)

    h1 = jnp.concatenate([g[...] for g in hrefs[0:4]], axis=1)
    h2 = jnp.concatenate([g[...] for g in hrefs[4:8]], axis=1)
    h3 = jnp.concatenate([g[...] for g in hrefs[8:12]], axis=1)
    z0 = (_dot(h1, w0_ref[0:H])
          + _dot(h2, w0_ref[H:2 * H])
          + _dot(h3, w0_ref[2 * H:3 * H])
          + b0_ref[...])
    blk = _relu(z0)
    blk2 = _relu(_dot(blk, w1_ref[...])
                 + b1_ref[...])
    zs = (_dot(h1, ws_ref[0:H])
          + _dot(h2, ws_ref[H:2 * H])
          + _dot(h3, ws_ref[2 * H:3 * H])
          + bs_ref[...])
    l_enc = blk2 + zs

    res = _dotg(l_enc, g_ref[...], (((1,), (1,)), ((), ())))
    seg = b_ref[...]
    gid = lax.broadcasted_iota(jnp.int32, (RB, G), 1)
    posm = seg == gid
    rowid = lax.broadcasted_iota(jnp.int32, (RB, G), 0) + i * RB
    valid = rowid < N
    sp = jnp.maximum(-res, 0.0) + jnp.log1p(jnp.exp(-jnp.abs(res)))
    posv = jnp.where(posm, LOG2 - sp, 0.0)
    negv = jnp.where(jnp.logical_and(jnp.logical_not(posm), valid),
                     sp + res - LOG2, 0.0)
    pos_ref[...] += jnp.sum(posv).reshape(1, 1)
    neg_ref[...] += jnp.sum(negv).reshape(1, 1)


def _loss_stage(batch_p, h1g, h2g, h3g, g_enc, lW0, lb0, lW1, lb1, lWs, lbs):
    gspec = pl.BlockSpec((RB, GCOL), lambda i: (i, 0))
    pos, neg = pl.pallas_call(
        _loss_body,
        grid=(NBLK,),
        in_specs=[pl.BlockSpec((RB, 1), lambda i: (i, 0))] + [gspec] * 12 + [
            pl.BlockSpec((G, H), lambda i: (0, 0)),
            pl.BlockSpec((3 * H, H), lambda i: (0, 0)),
            pl.BlockSpec((1, H), lambda i: (0, 0)),
            pl.BlockSpec((H, H), lambda i: (0, 0)),
            pl.BlockSpec((1, H), lambda i: (0, 0)),
            pl.BlockSpec((3 * H, H), lambda i: (0, 0)),
            pl.BlockSpec((1, H), lambda i: (0, 0)),
        ],
        out_specs=[pl.BlockSpec((1, 1), lambda i: (0, 0))] * 2,
        out_shape=[jax.ShapeDtypeStruct((1, 1), jnp.float32)] * 2,
    )(batch_p, *h1g, *h2g, *h3g, g_enc, lW0, lb0.reshape(1, H), lW1,
      lb1.reshape(1, H), lWs, lbs.reshape(1, H))
    return pos, neg


IDXH = ECH // 2


def _make_sc_agg(ncg):
    gpc = ncg // NSC
    mesh = plsc.VectorSubcoreMesh(core_axis_name="c", subcore_axis_name="s")

    def body(*refs):
        hgs = refs[0:ncg]
        src_h = refs[ncg]
        dst_h = refs[ncg + 1]
        outs = refs[ncg + 2:2 * ncg + 2]
        (src_v, dst_v, buf_a, buf_b, agg_sh,
         gsem_a, gsem_b, ssem_a, ssem_b) = refs[2 * ncg + 2:]
        bufs = (buf_a, buf_b)
        gsems = (gsem_a, gsem_b)
        ssems = (ssem_a, ssem_b)
        c = lax.axis_index("c")
        s = lax.axis_index("s")
        base = s * RPT

        def gstart(gi, j, b):
            pltpu.async_copy(hgs[gi].at[src_v.at[j]], bufs[b], gsems[b])

        def gwait(gi, j, b):
            pltpu.make_async_copy(hgs[gi].at[src_v.at[j]], bufs[b],
                                  gsems[b]).wait()

        def sstart(j, b):
            pltpu.async_copy(bufs[b], agg_sh.at[dst_v.at[j]], ssems[b],
                             add=True)

        def swait(j, b):
            pltpu.make_async_copy(bufs[b], agg_sh.at[dst_v.at[j]],
                                  ssems[b]).wait()

        for gl in range(gpc):
            for ci in range(NSC):
                @pl.when(c == ci)
                def _(gi=ci * gpc + gl):
                    pltpu.sync_copy(hgs[gi].at[pl.ds(base, RPT)],
                                    agg_sh.at[pl.ds(base, RPT)])
            plsc.subcore_barrier()
            for ci in range(NSC):
                @pl.when(c == ci)
                def _(gi=ci * gpc + gl):
                    for half in range(2):
                        pltpu.sync_copy(src_h.at[s, pl.ds(half * IDXH, IDXH)],
                                        src_v)
                        pltpu.sync_copy(dst_h.at[s, pl.ds(half * IDXH, IDXH)],
                                        dst_v)
                        gstart(gi, 0, 0)

                        def step(jj, carry, gi=gi):
                            for b in range(2):
                                j = jj * 2 + b
                                gwait(gi, j, b)
                                sstart(j, b)

                                @pl.when(j + 1 < IDXH)
                                def _(b=b, j=j, gi=gi):
                                    @pl.when(j >= 1)
                                    def _():
                                        swait(j - 1, 1 - b)
                                    gstart(gi, j + 1, 1 - b)
                            return carry
                        lax.fori_loop(0, IDXH // 2, step, 0, unroll=False)
                        swait(IDXH - 2, IDXH % 2)
                        swait(IDXH - 1, (IDXH - 1) % 2)
            plsc.subcore_barrier()
            for ci in range(NSC):
                @pl.when(c == ci)
                def _(gi=ci * gpc + gl):
                    pltpu.sync_copy(agg_sh.at[pl.ds(base, RPT)],
                                    outs[gi].at[pl.ds(base, RPT)])

    return pl.kernel(
        body,
        mesh=mesh,
        out_type=[jax.ShapeDtypeStruct((NPAD, GCOL), jnp.float32)] * ncg,
        scratch_types=[
            pltpu.VMEM((IDXH, 128), jnp.int32),
            pltpu.VMEM((IDXH, 128), jnp.int32),
            pltpu.VMEM((128, GCOL), jnp.float32),
            pltpu.VMEM((128, GCOL), jnp.float32),
            pltpu.VMEM_SHARED((NPAD, GCOL), jnp.float32),
            pltpu.SemaphoreType.DMA,
            pltpu.SemaphoreType.DMA,
            pltpu.SemaphoreType.DMA,
            pltpu.SemaphoreType.DMA,
        ],
    )


_sc_agg = {2: _make_sc_agg(2), 4: _make_sc_agg(4)}


def _edge_agg(groups, srcp, dstp):
    return list(_sc_agg[len(groups)](*groups, srcp, dstp))


def kernel(x, edge_index, batch, W0a, b0a, W0b, b0b, W1a, b1a, W1b, b1b,
           W2a, b2a, W2b, b2b, lW0, lb0, lW1, lb1, lWs, lbs,
           gW0, gb0, gW1, gb1, gWs, gbs):
    src = edge_index[0]
    dst = edge_index[1]
    pad_e = EPAD - E
    srcp = jnp.concatenate(
        [src.astype(jnp.int32),
         N + (jnp.arange(pad_e, dtype=jnp.int32) % 8)]).reshape(NTILE, ECH, 128)
    dstp = jnp.concatenate(
        [dst.astype(jnp.int32),
         N + (jnp.arange(pad_e, dtype=jnp.int32) % (NPAD - N))]
    ).reshape(NTILE, ECH, 128)
    batch_p = jnp.pad(batch, (0, NPAD - N), constant_values=G)
    batch_p = batch_p.astype(jnp.int32).reshape(NPAD, 1)
    xg = [jnp.pad(x[:, i * GCOL:(i + 1) * GCOL], ((0, NPAD - N), (0, 0)))
          for i in range(DIN // GCOL)]

    h1 = _gin_mlp(_edge_agg(xg, srcp, dstp), W0a, b0a, W0b, b0b)
    h2 = _gin_mlp(_edge_agg(h1, srcp, dstp), W1a, b1a, W1b, b1b)
    h3 = _gin_mlp(_edge_agg(h2, srcp, dstp), W2a, b2a, W2b, b2b)

    g1, g2, g3 = _segment_readout(batch_p, h1, h2, h3)
    g_enc = _global_ff(g1, g2, g3, gW0, gb0, gW1, gb1, gWs, gbs)
    pos, neg = _loss_stage(batch_p, h1, h2, h3, g_enc,
                           lW0, lb0, lW1, lb1, lWs, lbs)
    e_pos = pos[0, 0] / jnp.float32(N)
    e_neg = neg[0, 0] / jnp.float32(N * (G - 1))
    return e_neg - e_pos
